# Initial kernel scaffold; baseline (speedup 1.0000x reference)
import jax
import jax.numpy as jnp
from jax.experimental import pallas as pl


def kernel(labels, outputs, features, train_step, epoch):
    raise NotImplementedError("write your pallas kernel here")


# speedup vs baseline: 1.2749x; 1.2749x over previous
import jax
import jax.numpy as jnp
from jax.experimental import pallas as pl
from jax.experimental.pallas import tpu as pltpu

_N = 4608
_C = 256
_BM = 576
_BN = 512
_NB = _N // _BM
_NCH = _N // _BN
_LANES = 128


def _snnl_rows_kernel(xi_ref, xall_ref, yrow_ref, ylane_ref, out_ref):
    i0 = pl.program_id(0) * _BM
    xi = xi_ref[...]
    sq_i = jnp.sum(xi * xi, axis=1, keepdims=True)
    yr = jnp.broadcast_to(yrow_ref[...], (_BM, _BN))
    rc = (jax.lax.broadcasted_iota(jnp.int32, (_BM, _BN), 0) + i0
          - jax.lax.broadcasted_iota(jnp.int32, (_BM, _BN), 1))
    ones8 = jnp.ones((8, _C), dtype=jnp.float32)
    acc_n = jnp.zeros((_BM, _LANES), dtype=jnp.float32)
    acc_d = jnp.zeros((_BM, _LANES), dtype=jnp.float32)
    for j in range(_NCH):
        j0 = j * _BN
        xj = xall_ref[j0:j0 + _BN, :]
        dotv = jax.lax.dot_general(
            xi, xj, (((1,), (1,)), ((), ())),
            preferred_element_type=jnp.float32)
        sq_j = jax.lax.dot_general(
            ones8, xj * xj, (((1,), (1,)), ((), ())),
            preferred_element_type=jnp.float32)[0:1, :]
        d = jnp.maximum(sq_i + sq_j - 2.0 * dotv, 0.0)
        e = jnp.exp(-d)
        e_den = jnp.where(rc == j0, 0.0, e)
        e_num = jnp.where(yr == ylane_ref[0:1, j0:j0 + _BN], e_den, 0.0)
        for k in range(_BN // _LANES):
            s = slice(k * _LANES, (k + 1) * _LANES)
            acc_d = acc_d + e_den[:, s]
            acc_n = acc_n + e_num[:, s]
    num = jnp.sum(acc_n, axis=1, keepdims=True)
    den = jnp.sum(acc_d, axis=1, keepdims=True)
    out_ref[...] = jnp.log(num / den)


def _mean_kernel(v_ref, o_ref):
    s = jnp.sum(v_ref[...], axis=1, keepdims=True)
    o_ref[...] = s * (-1.0 / _N)


def kernel(labels, outputs, features, train_step, epoch):
    y = labels[:, ::8, ::8].reshape(-1).astype(jnp.int32)
    x = features.reshape(-1, _C).astype(jnp.float32)
    logr = pl.pallas_call(
        _snnl_rows_kernel,
        grid=(_NB,),
        in_specs=[
            pl.BlockSpec((_BM, _C), lambda i: (i, 0)),
            pl.BlockSpec((_N, _C), lambda i: (0, 0)),
            pl.BlockSpec((_BM, 1), lambda i: (i, 0)),
            pl.BlockSpec((1, _N), lambda i: (0, 0)),
        ],
        out_specs=pl.BlockSpec((_BM, 1), lambda i: (i, 0)),
        out_shape=jax.ShapeDtypeStruct((_N, 1), jnp.float32),
        compiler_params=pltpu.CompilerParams(
            dimension_semantics=("parallel",),
        ),
        name="snnl_rows",
    )(x, x, y.reshape(_N, 1), y.reshape(1, _N))
    loss = pl.pallas_call(
        _mean_kernel,
        out_shape=jax.ShapeDtypeStruct((1, 1), jnp.float32),
        name="snnl_mean",
    )(logr.reshape(1, _N))
    return loss.reshape(())
